# baseline (speedup 1.0000x reference)
import functools

import jax
import jax.numpy as jnp
from jax import lax
from jax.experimental import pallas as pl
from jax.experimental.pallas import tpu as pltpu
from jax.experimental.pallas import tpu_sc as plsc

N = 50000
HID = 128
HEADS = 8
DH = 16
E = 200000

NWORK = 32
EPW = 6656
EPAD = NWORK * EPW
GCHUNK = 128
NGCH = EPW // GCHUNK

CB = 256
NBLK = EPAD // CB
NSEG_PAD = 50688
OH = 272

NODE_PAD = 50176
NPW = NODE_PAD // NWORK
SCHUNK = 392
NSCH = NPW // SCHUNK
NCOL = 144


def _prep_edges(ei):
    src, dst = ei[0], ei[1]
    order = jnp.argsort(dst)
    srcs = src[order]
    dsts = dst[order]
    srcp = jnp.concatenate([srcs, jnp.zeros((EPAD - E,), jnp.int32)])
    dstp = jnp.concatenate([dsts, jnp.full((EPAD - E,), dsts[-1], jnp.int32)])
    bnd = jnp.concatenate(
        [jnp.zeros((1,), jnp.int32), (dstp[1:] != dstp[:-1]).astype(jnp.int32)])
    seg = jnp.cumsum(bnd, dtype=jnp.int32)
    first_seg = seg[::CB]
    lseg3 = seg.reshape(NBLK, 1, CB)
    seg_of_node = jnp.full((NODE_PAD,), NSEG_PAD - 1, jnp.int32)
    seg_of_node = seg_of_node.at[dstp].set(seg)
    node_of_seg = jnp.zeros((NSEG_PAD,), jnp.int32).at[seg].set(dstp)
    return srcp, lseg3, first_seg, seg_of_node, node_of_seg


def _block_diag(m):
    out = jnp.zeros((HID, HID), jnp.float32)
    for h in range(HEADS):
        out = out.at[h * DH:(h + 1) * DH, h * DH:(h + 1) * DH].set(m[h])
    return out


def _fold_layer(lp):
    w = {}
    for nt in ("author", "paper"):
        W = lp["w_kqv"][nt]
        b = lp["b_kqv"][nt]
        w[nt] = dict(
            Wk=W[:, :HID], Wq=W[:, HID:2 * HID], Wv=W[:, 2 * HID:],
            bk=b[:HID], bq=b[HID:2 * HID], bv=b[2 * HID:])
    rel_kv = {}
    for rel, src_nt in (("writes", "author"), ("rev_writes", "paper"),
                        ("cites", "paper")):
        scale = lp["p_rel"][rel] / jnp.sqrt(jnp.float32(DH))
        bda = _block_diag(lp["a_rel"][rel] * scale[:, None, None])
        bdm = _block_diag(lp["m_rel"][rel])
        s = w[src_nt]
        Wkv = jnp.concatenate([s["Wk"] @ bda, s["Wv"] @ bdm], axis=1)
        bkv = jnp.concatenate([s["bk"] @ bda, s["bv"] @ bdm])
        rel_kv[rel] = (Wkv, bkv)
    Wa = jnp.concatenate([w["author"]["Wq"], rel_kv["writes"][0]], axis=1)
    ba = jnp.concatenate([w["author"]["bq"], rel_kv["writes"][1]])[None, :]
    Wp = jnp.concatenate([w["paper"]["Wq"], rel_kv["rev_writes"][0],
                          rel_kv["cites"][0]], axis=1)
    bp = jnp.concatenate([w["paper"]["bq"], rel_kv["rev_writes"][1],
                          rel_kv["cites"][1]])[None, :]
    epi = {}
    for nt in ("author", "paper"):
        a = jax.nn.sigmoid(lp["skip"][nt])
        epi[nt] = (lp["w_out"][nt] * a, (lp["b_out"][nt] * a)[None, :],
                   jnp.reshape(1.0 - a, (1, 1)))
    return Wa, ba, Wp, bp, epi


def _proj(x, W, b, widths):
    M = x.shape[0]
    F = W.shape[1]
    RB = 512
    grid = pl.cdiv(M, RB)

    def body(x_ref, w_ref, b_ref, *out_refs):
        h = jnp.dot(x_ref[...], w_ref[...],
                    preferred_element_type=jnp.float32) + b_ref[...]
        ofs = 0
        for r, wd in zip(out_refs, widths):
            r[...] = h[:, ofs:ofs + wd]
            ofs += wd

    return pl.pallas_call(
        body,
        grid=(grid,),
        in_specs=[
            pl.BlockSpec((RB, HID), lambda i: (i, 0)),
            pl.BlockSpec((HID, F), lambda i: (0, 0)),
            pl.BlockSpec((1, F), lambda i: (0, 0)),
        ],
        out_specs=[pl.BlockSpec((RB, wd), lambda i: (i, 0)) for wd in widths],
        out_shape=[jax.ShapeDtypeStruct((M, wd), jnp.float32) for wd in widths],
    )(x, W, b)


QPW = NSEG_PAD // NWORK
QCHUNKS = [GCHUNK] * 12 + [48]


def _gather_type(kv_tab, q_tab, srcp, node_of_seg):
    mesh = plsc.VectorSubcoreMesh(core_axis_name="c", subcore_axis_name="s")

    @functools.partial(
        pl.kernel, mesh=mesh,
        out_type=[jax.ShapeDtypeStruct((EPAD, 2 * HID), jnp.float32),
                  jax.ShapeDtypeStruct((NSEG_PAD, HID), jnp.float32)],
        scratch_types=(
            [pltpu.VMEM((EPW,), jnp.int32), pltpu.VMEM((QPW,), jnp.int32)]
            + [pltpu.VMEM((GCHUNK, 2 * HID), jnp.float32)] * 2
            + [pltpu.VMEM((GCHUNK, HID), jnp.float32)] * 2
            + [pltpu.SemaphoreType.DMA] * 8
        ))
    def k(kv_hbm, q_hbm, src_hbm, nos_hbm, kve_hbm, qseg_hbm,
          src_v, nos_v, kv0, kv1, q0, q1,
          gk0, gk1, wk0, wk1, gq0, gq1, wq0, wq1):
        wid = lax.axis_index("s") * 2 + lax.axis_index("c")
        base = wid * EPW
        qbase = wid * QPW
        pltpu.sync_copy(src_hbm.at[pl.ds(base, EPW)], src_v)
        pltpu.sync_copy(nos_hbm.at[pl.ds(qbase, QPW)], nos_v)
        kvb = (kv0, kv1)
        qb = (q0, q1)
        gks = (gk0, gk1)
        wks = (wk0, wk1)
        gqs = (gq0, gq1)
        wqs = (wq0, wq1)

        def g_start(c, p):
            sl = pl.ds(c * GCHUNK, GCHUNK)
            pltpu.async_copy(kv_hbm.at[src_v.at[sl]], kvb[p], gks[p])

        def g_wait(p):
            z = pl.ds(0, GCHUNK)
            pltpu.make_async_copy(kv_hbm.at[src_v.at[z]], kvb[p],
                                  gks[p]).wait()

        def w_start(c, p):
            osl = pl.ds(base + c * GCHUNK, GCHUNK)
            pltpu.async_copy(kvb[p], kve_hbm.at[osl], wks[p])

        def w_wait(p):
            z = pl.ds(0, GCHUNK)
            pltpu.make_async_copy(kvb[p], kve_hbm.at[z], wks[p]).wait()

        g_start(0, 0)

        @pl.loop(0, NGCH // 2)
        def _(i):
            @pl.when(i > 0)
            def _():
                w_wait(1)

            g_wait(0)
            g_start(i * 2 + 1, 1)
            w_start(i * 2, 0)
            w_wait(0)
            g_wait(1)

            @pl.when(i < NGCH // 2 - 1)
            def _():
                g_start(i * 2 + 2, 0)

            w_start(i * 2 + 1, 1)

        w_wait(1)

        qoff = [0]
        for sz in QCHUNKS:
            qoff.append(qoff[-1] + sz)

        def qg_start(n, p):
            sl = pl.ds(qoff[n], QCHUNKS[n])
            dst = q0.at[pl.ds(0, QCHUNKS[n])] if p == 0 \
                else q1.at[pl.ds(0, QCHUNKS[n])]
            return pltpu.async_copy(q_hbm.at[nos_v.at[sl]], dst, gqs[p])

        def qw_start(n, p):
            src = q0.at[pl.ds(0, QCHUNKS[n])] if p == 0 \
                else q1.at[pl.ds(0, QCHUNKS[n])]
            osl = pl.ds(qbase + qoff[n], QCHUNKS[n])
            return pltpu.async_copy(src, qseg_hbm.at[osl], wqs[p])

        nq = len(QCHUNKS)
        pg = [None, None]
        pw = [None, None]
        pg[0] = qg_start(0, 0)
        for n in range(nq):
            p = n % 2
            pn = (n + 1) % 2
            if n > 0 and pw[pn] is not None:
                pw[pn].wait()
            pg[p].wait()
            if n + 1 < nq:
                pg[pn] = qg_start(n + 1, pn)
            pw[p] = qw_start(n, p)
        pw[(nq - 1) % 2].wait()

    return k(kv_tab, q_tab, srcp, node_of_seg)


def _seg_softmax(kve, qseg, lseg3, first_seg):
    def body(fs_ref, kv_ref, qs_hbm, ls_ref, num_ref, dent_ref, qs_ref, qsem):
        i = pl.program_id(0)

        @pl.when(i == 0)
        def _():
            num_ref[...] = jnp.zeros_like(num_ref)
            dent_ref[...] = jnp.zeros_like(dent_ref)
            pltpu.make_async_copy(qs_hbm, qs_ref, qsem).start()
            pltpu.make_async_copy(qs_hbm, qs_ref, qsem).wait()

        sel = (jax.lax.broadcasted_iota(jnp.int32, (HID, HEADS), 0) // DH
               == jax.lax.broadcasted_iota(jnp.int32, (HID, HEADS), 1)
               ).astype(jnp.float32)
        ke = kv_ref[:, :HID]
        ve = kv_ref[:, HID:]
        fs = fs_ref[i]
        fsa = (fs // 8) * 8
        loc = ls_ref[0, 0, :] - fsa
        oht = (jax.lax.broadcasted_iota(jnp.int32, (CB, OH), 1)
               == loc[:, None]).astype(jnp.float32)
        qe = jnp.dot(oht, qs_ref[pl.ds(fsa, OH), :],
                     preferred_element_type=jnp.float32)
        prod = qe * ke
        alpha = jnp.dot(prod, sel, preferred_element_type=jnp.float32)
        ex = jnp.exp(alpha)
        eid = jax.lax.broadcasted_iota(jnp.int32, (CB, HEADS), 0) + i * CB
        ex = jnp.where(eid < E, ex, 0.0)
        exe = jnp.dot(ex, sel.T, preferred_element_type=jnp.float32)
        vals = ve * exe
        oh = (jax.lax.broadcasted_iota(jnp.int32, (OH, CB), 0)
              == loc[None, :]).astype(jnp.float32)
        pnum = jnp.dot(oh, vals, preferred_element_type=jnp.float32)
        fsd = (fs // 128) * 128
        locd = ls_ref[0, 0, :] - fsd
        ohd = (jax.lax.broadcasted_iota(jnp.int32, (CB, 512), 1)
               == locd[:, None]).astype(jnp.float32)
        pdent = jax.lax.dot_general(
            ex, ohd, (((0,), (0,)), ((), ())),
            preferred_element_type=jnp.float32)
        num_ref[pl.ds(fsa, OH), :] += pnum
        dent_ref[:, pl.ds(fsd, 512)] += pdent

    grid_spec = pltpu.PrefetchScalarGridSpec(
        num_scalar_prefetch=1,
        grid=(NBLK,),
        in_specs=[
            pl.BlockSpec((CB, 2 * HID), lambda i, fs: (i, 0)),
            pl.BlockSpec(memory_space=pl.ANY),
            pl.BlockSpec((1, 1, CB), lambda i, fs: (i, 0, 0)),
        ],
        out_specs=[pl.BlockSpec((NSEG_PAD, HID), lambda i, fs: (0, 0)),
                   pl.BlockSpec((HEADS, NSEG_PAD), lambda i, fs: (0, 0))],
        scratch_shapes=[pltpu.VMEM((NSEG_PAD, HID), jnp.float32),
                        pltpu.SemaphoreType.DMA],
    )
    return pl.pallas_call(
        body,
        grid_spec=grid_spec,
        out_shape=[jax.ShapeDtypeStruct((NSEG_PAD, HID), jnp.float32),
                   jax.ShapeDtypeStruct((HEADS, NSEG_PAD), jnp.float32)],
    )(first_seg, kve, qseg, lseg3)


def _normalize(num, den):
    DB = NSEG_PAD // 32

    def body(n_ref, d_ref, o_ref):
        sel = (jax.lax.broadcasted_iota(jnp.int32, (HID, HEADS), 0) // DH
               == jax.lax.broadcasted_iota(jnp.int32, (HID, HEADS), 1)
               ).astype(jnp.float32)
        dexp = jnp.dot(d_ref[...], sel.T, preferred_element_type=jnp.float32)
        o_ref[...] = n_ref[...] / (dexp + 1e-16)

    return pl.pallas_call(
        body,
        grid=(32,),
        in_specs=[pl.BlockSpec((DB, HID), lambda i: (i, 0)),
                  pl.BlockSpec((DB, HEADS), lambda i: (i, 0))],
        out_specs=pl.BlockSpec((DB, HID), lambda i: (i, 0)),
        out_shape=jax.ShapeDtypeStruct((NSEG_PAD, HID), jnp.float32),
    )(num, den)


def _seg_to_node_all(rows3, son3):
    mesh = plsc.VectorSubcoreMesh(core_axis_name="c", subcore_axis_name="s")
    os_ = jax.ShapeDtypeStruct((NODE_PAD, HID), jnp.float32)

    @functools.partial(
        pl.kernel, mesh=mesh,
        out_type=[os_, os_, os_],
        scratch_types=(
            [pltpu.VMEM((NPW,), jnp.int32)] * 3
            + [pltpu.VMEM((SCHUNK, HID), jnp.float32)] * 2
            + [pltpu.SemaphoreType.DMA] * 5
        ))
    def k(rows_w, rows_r, rows_c, son_w, son_r, son_c, out_w, out_r, out_c,
          ix_w, ix_r, ix_c, b0, b1, g0, g1, w0, w1, ixs):
        wid = lax.axis_index("s") * 2 + lax.axis_index("c")
        base = wid * NPW
        rowst = (rows_w, rows_r, rows_c)
        sont = (son_w, son_r, son_c)
        outt = (out_w, out_r, out_c)
        ixb = (ix_w, ix_r, ix_c)
        bufs = (b0, b1)
        gsem = (g0, g1)
        wsem = (w0, w1)
        for t in range(3):
            pltpu.async_copy(sont[t].at[pl.ds(base, NPW)], ixb[t], ixs).wait()

        def start_gather(t, c):
            sl = pl.ds(c * SCHUNK, SCHUNK)
            return pltpu.async_copy(rowst[t].at[ixb[t].at[sl]], bufs[c % 2],
                                    gsem[c % 2])

        chunks = [(t, c) for t in range(3) for c in range(NSCH)]
        pend_g = start_gather(0, 0)
        pend_w = None
        for n, (t, c) in enumerate(chunks):
            if pend_w is not None:
                pend_w.wait()
            pend_g.wait()
            if n + 1 < len(chunks):
                tn, cn = chunks[n + 1]
                pend_g = start_gather(tn, cn)
            pend_w = pltpu.async_copy(
                bufs[c % 2], outt[t].at[pl.ds(base + c * SCHUNK, SCHUNK)],
                wsem[c % 2])
        pend_w.wait()

    return k(*rows3, *son3)


def _epilogue(x, bufs, Wo, bo, sscal):
    M = x.shape[0]
    RB = 512
    grid = pl.cdiv(M, RB)
    nb = len(bufs)

    def body(*refs):
        x_ref = refs[0]
        brefs = refs[1:1 + nb]
        w_ref, b_ref, s_ref, o_ref = refs[1 + nb:]
        o = brefs[0][:, :HID]
        for br in brefs[1:]:
            o = o + br[:, :HID]
        g = jax.nn.gelu(o)
        o_ref[...] = (jnp.dot(g, w_ref[...],
                              preferred_element_type=jnp.float32)
                      + b_ref[...] + s_ref[0, 0] * x_ref[...])

    return pl.pallas_call(
        body,
        grid=(grid,),
        in_specs=(
            [pl.BlockSpec((RB, HID), lambda i: (i, 0))]
            + [pl.BlockSpec((RB, HID), lambda i: (i, 0))] * nb
            + [pl.BlockSpec((HID, HID), lambda i: (0, 0)),
               pl.BlockSpec((1, HID), lambda i: (0, 0)),
               pl.BlockSpec(memory_space=pltpu.SMEM)]
        ),
        out_specs=pl.BlockSpec((RB, HID), lambda i: (i, 0)),
        out_shape=jax.ShapeDtypeStruct((M, HID), jnp.float32),
    )(x, *bufs, Wo, bo, sscal)


def kernel(x_author, x_paper, ei_writes, ei_rev_writes, ei_cites, params):
    preps = {
        "writes": _prep_edges(ei_writes),
        "rev_writes": _prep_edges(ei_rev_writes),
        "cites": _prep_edges(ei_cites),
    }
    rel_nt = {"writes": ("author", "paper"),
              "rev_writes": ("paper", "author"),
              "cites": ("paper", "paper")}
    x = {"author": x_author, "paper": x_paper}
    for lp in params:
        Wa, ba, Wp, bp, epi = _fold_layer(lp)
        qa, kv_w = _proj(x["author"], Wa, ba, [HID, 2 * HID])
        qp, kv_r, kv_c = _proj(x["paper"], Wp, bp, [HID, 2 * HID, 2 * HID])
        q = {"author": qa, "paper": qp}
        kv = {"writes": kv_w, "rev_writes": kv_r, "cites": kv_c}
        rels = ("writes", "rev_writes", "cites")
        segrows = []
        for r in rels:
            srcp, lseg3, first_seg, _, node_of_seg = preps[r]
            kve, qseg = _gather_type(kv[r], q[rel_nt[r][1]], srcp, node_of_seg)
            num, dent = _seg_softmax(kve, qseg, lseg3, first_seg)
            segrows.append(_normalize(num, jnp.swapaxes(dent, 0, 1)))
        outs = _seg_to_node_all(segrows, tuple(preps[r][3] for r in rels))
        buf = dict(zip(rels, outs))
        x = {
            "author": _epilogue(x["author"], [buf["rev_writes"]], *epi["author"]),
            "paper": _epilogue(x["paper"], [buf["writes"], buf["cites"]],
                               *epi["paper"]),
        }
    return (x["author"], x["paper"])


# speedup vs baseline: 1.2568x; 1.2568x over previous
import functools

import jax
import jax.numpy as jnp
from jax import lax
from jax.experimental import pallas as pl
from jax.experimental.pallas import tpu as pltpu
from jax.experimental.pallas import tpu_sc as plsc

N = 50000
HID = 128
HEADS = 8
DH = 16
E = 200000

NWORK = 32
EPW = 6528
EPAD = NWORK * EPW
GCHUNK = 128
NGCH = EPW // GCHUNK

CB = 256
NBLK = EPAD // CB
NSEG_PAD = 50688
OH = 272

NODE_PAD = 50176
NPW = NODE_PAD // NWORK
SCHUNK = 392
NSCH = NPW // SCHUNK


def _prep_edges(ei):
    src, dst = ei[0], ei[1]
    order = jnp.argsort(dst)
    srcs = src[order]
    dsts = dst[order]
    srcp = jnp.concatenate([srcs, jnp.zeros((EPAD - E,), jnp.int32)])
    dstp = jnp.concatenate([dsts, jnp.full((EPAD - E,), dsts[-1], jnp.int32)])
    bnd = jnp.concatenate(
        [jnp.zeros((1,), jnp.int32), (dstp[1:] != dstp[:-1]).astype(jnp.int32)])
    seg = jnp.cumsum(bnd, dtype=jnp.int32)
    first_seg = seg[::CB]
    lseg3 = seg.reshape(NBLK, 1, CB)
    seg_of_node = jnp.full((NODE_PAD,), NSEG_PAD - 1, jnp.int32)
    seg_of_node = seg_of_node.at[dstp].set(seg)
    return srcp, dstp, lseg3, first_seg, seg_of_node


def _block_diag(m):
    out = jnp.zeros((HID, HID), jnp.float32)
    for h in range(HEADS):
        out = out.at[h * DH:(h + 1) * DH, h * DH:(h + 1) * DH].set(m[h])
    return out


def _fold_layer(lp):
    w = {}
    for nt in ("author", "paper"):
        W = lp["w_kqv"][nt]
        b = lp["b_kqv"][nt]
        w[nt] = dict(
            Wk=W[:, :HID], Wq=W[:, HID:2 * HID], Wv=W[:, 2 * HID:],
            bk=b[:HID], bq=b[HID:2 * HID], bv=b[2 * HID:])
    rel_kv = {}
    for rel, src_nt in (("writes", "author"), ("rev_writes", "paper"),
                        ("cites", "paper")):
        scale = lp["p_rel"][rel] / jnp.sqrt(jnp.float32(DH))
        bda = _block_diag(lp["a_rel"][rel] * scale[:, None, None])
        bdm = _block_diag(lp["m_rel"][rel])
        s = w[src_nt]
        Wkv = jnp.concatenate([s["Wk"] @ bda, s["Wv"] @ bdm], axis=1)
        bkv = jnp.concatenate([s["bk"] @ bda, s["bv"] @ bdm])
        rel_kv[rel] = (Wkv, bkv)
    Wa = jnp.concatenate([w["author"]["Wq"], rel_kv["writes"][0]], axis=1)
    ba = jnp.concatenate([w["author"]["bq"], rel_kv["writes"][1]])[None, :]
    Wp = jnp.concatenate([w["paper"]["Wq"], rel_kv["rev_writes"][0],
                          rel_kv["cites"][0]], axis=1)
    bp = jnp.concatenate([w["paper"]["bq"], rel_kv["rev_writes"][1],
                          rel_kv["cites"][1]])[None, :]
    epi = {}
    for nt in ("author", "paper"):
        a = jax.nn.sigmoid(lp["skip"][nt])
        epi[nt] = (lp["w_out"][nt] * a, (lp["b_out"][nt] * a)[None, :],
                   jnp.reshape(1.0 - a, (1, 1)))
    return Wa, ba, Wp, bp, epi


def _proj(x, W, b, widths):
    M = x.shape[0]
    F = W.shape[1]
    RB = 512
    grid = pl.cdiv(M, RB)

    def body(x_ref, w_ref, b_ref, *out_refs):
        h = jnp.dot(x_ref[...], w_ref[...],
                    preferred_element_type=jnp.float32) + b_ref[...]
        ofs = 0
        for r, wd in zip(out_refs, widths):
            r[...] = h[:, ofs:ofs + wd]
            ofs += wd

    return pl.pallas_call(
        body,
        grid=(grid,),
        in_specs=[
            pl.BlockSpec((RB, HID), lambda i: (i, 0)),
            pl.BlockSpec((HID, F), lambda i: (0, 0)),
            pl.BlockSpec((1, F), lambda i: (0, 0)),
        ],
        out_specs=[pl.BlockSpec((RB, wd), lambda i: (i, 0)) for wd in widths],
        out_shape=[jax.ShapeDtypeStruct((M, wd), jnp.float32) for wd in widths],
    )(x, W, b)


def _edge_gather(kv_tab, q_tab, srcp, dstp):
    mesh = plsc.VectorSubcoreMesh(core_axis_name="c", subcore_axis_name="s")

    @functools.partial(
        pl.kernel, mesh=mesh,
        out_type=[jax.ShapeDtypeStruct((EPAD, 2 * HID), jnp.float32),
                  jax.ShapeDtypeStruct((EPAD, HID), jnp.float32)],
        scratch_types=[
            pltpu.VMEM((EPW,), jnp.int32),
            pltpu.VMEM((EPW,), jnp.int32),
            pltpu.VMEM((GCHUNK, 2 * HID), jnp.float32),
            pltpu.VMEM((GCHUNK, 2 * HID), jnp.float32),
            pltpu.VMEM((GCHUNK, HID), jnp.float32),
            pltpu.VMEM((GCHUNK, HID), jnp.float32),
            pltpu.SemaphoreType.DMA,
            pltpu.SemaphoreType.DMA,
            pltpu.SemaphoreType.DMA,
            pltpu.SemaphoreType.DMA,
        ])
    def k(kv_hbm, q_hbm, src_hbm, dst_hbm, kve_hbm, qe_hbm,
          src_v, dst_v, kv0, kv1, q0, q1, sk0, sk1, sq0, sq1):
        wid = lax.axis_index("s") * 2 + lax.axis_index("c")
        base = wid * EPW
        pltpu.sync_copy(src_hbm.at[pl.ds(base, EPW)], src_v)
        pltpu.sync_copy(dst_hbm.at[pl.ds(base, EPW)], dst_v)
        kvb = (kv0, kv1)
        qb = (q0, q1)
        ksem = (sk0, sk1)
        qsem = (sq0, sq1)

        def start(c):
            sl = pl.ds(c * GCHUNK, GCHUNK)
            hk = pltpu.async_copy(kv_hbm.at[src_v.at[sl]], kvb[c % 2],
                                  ksem[c % 2])
            hq = pltpu.async_copy(q_hbm.at[dst_v.at[sl]], qb[c % 2],
                                  qsem[c % 2])
            return hk, hq

        pending = start(0)
        for c in range(NGCH):
            hk, hq = pending
            hk.wait()
            hq.wait()
            if c + 1 < NGCH:
                pending = start(c + 1)
            osl = pl.ds(base + c * GCHUNK, GCHUNK)
            pltpu.sync_copy(kvb[c % 2], kve_hbm.at[osl])
            pltpu.sync_copy(qb[c % 2], qe_hbm.at[osl])

    return k(kv_tab, q_tab, srcp, dstp)


def _seg_softmax(kve, qe, lseg3, first_seg):
    def body(fs_ref, kv_ref, q_ref, ls_ref, num_ref, den_ref):
        i = pl.program_id(0)

        @pl.when(i == 0)
        def _():
            num_ref[...] = jnp.zeros_like(num_ref)
            den_ref[...] = jnp.zeros_like(den_ref)

        sel = (jax.lax.broadcasted_iota(jnp.int32, (HID, HEADS), 0) // DH
               == jax.lax.broadcasted_iota(jnp.int32, (HID, HEADS), 1)
               ).astype(jnp.float32)
        ke = kv_ref[:, :HID]
        ve = kv_ref[:, HID:]
        prod = q_ref[...] * ke
        alpha = jnp.dot(prod, sel, preferred_element_type=jnp.float32)
        ex = jnp.exp(alpha)
        eid = jax.lax.broadcasted_iota(jnp.int32, (CB, HEADS), 0) + i * CB
        ex = jnp.where(eid < E, ex, 0.0)
        exe = jnp.dot(ex, sel.T, preferred_element_type=jnp.float32)
        vals = ve * exe
        fs = fs_ref[i]
        fsa = (fs // 8) * 8
        loc = ls_ref[0, 0, :] - fsa
        oh = (jax.lax.broadcasted_iota(jnp.int32, (OH, CB), 0)
              == loc[None, :]).astype(jnp.float32)
        pnum = jnp.dot(oh, vals, preferred_element_type=jnp.float32)
        pden = jnp.dot(oh, ex, preferred_element_type=jnp.float32)
        num_ref[pl.ds(fsa, OH), :] += pnum
        den_ref[pl.ds(fsa, OH), :] += pden

        @pl.when(i == NBLK - 1)
        def _():
            den = den_ref[...]
            dexp = jnp.dot(den, sel.T, preferred_element_type=jnp.float32)
            num_ref[...] = num_ref[...] / (dexp + 1e-16)

    grid_spec = pltpu.PrefetchScalarGridSpec(
        num_scalar_prefetch=1,
        grid=(NBLK,),
        in_specs=[
            pl.BlockSpec((CB, 2 * HID), lambda i, fs: (i, 0)),
            pl.BlockSpec((CB, HID), lambda i, fs: (i, 0)),
            pl.BlockSpec((1, 1, CB), lambda i, fs: (i, 0, 0)),
        ],
        out_specs=pl.BlockSpec((NSEG_PAD, HID), lambda i, fs: (0, 0)),
        scratch_shapes=[pltpu.VMEM((NSEG_PAD, HEADS), jnp.float32)],
    )
    return pl.pallas_call(
        body,
        grid_spec=grid_spec,
        out_shape=jax.ShapeDtypeStruct((NSEG_PAD, HID), jnp.float32),
    )(first_seg, kve, qe, lseg3)


def _seg_to_node(rows, seg_of_node):
    mesh = plsc.VectorSubcoreMesh(core_axis_name="c", subcore_axis_name="s")

    @functools.partial(
        pl.kernel, mesh=mesh,
        out_type=jax.ShapeDtypeStruct((NODE_PAD, HID), jnp.float32),
        scratch_types=[
            pltpu.VMEM((NPW,), jnp.int32),
            pltpu.VMEM((SCHUNK, HID), jnp.float32),
            pltpu.VMEM((SCHUNK, HID), jnp.float32),
            pltpu.SemaphoreType.DMA,
            pltpu.SemaphoreType.DMA,
        ])
    def k(rows_hbm, idx_hbm, out_hbm, idx_v, b0, b1, s0, s1):
        wid = lax.axis_index("s") * 2 + lax.axis_index("c")
        base = wid * NPW
        pltpu.sync_copy(idx_hbm.at[pl.ds(base, NPW)], idx_v)
        bufs = (b0, b1)
        sems = (s0, s1)

        def start(c):
            sl = pl.ds(c * SCHUNK, SCHUNK)
            return pltpu.async_copy(rows_hbm.at[idx_v.at[sl]], bufs[c % 2],
                                    sems[c % 2])

        pending = start(0)
        for c in range(NSCH):
            pending.wait()
            if c + 1 < NSCH:
                pending = start(c + 1)
            pltpu.sync_copy(bufs[c % 2],
                            out_hbm.at[pl.ds(base + c * SCHUNK, SCHUNK)])

    return k(rows, seg_of_node)


def _epilogue(x, bufs, Wo, bo, sscal):
    M = x.shape[0]
    RB = 512
    grid = pl.cdiv(M, RB)
    nb = len(bufs)

    def body(*refs):
        x_ref = refs[0]
        brefs = refs[1:1 + nb]
        w_ref, b_ref, s_ref, o_ref = refs[1 + nb:]
        o = brefs[0][...]
        for br in brefs[1:]:
            o = o + br[...]
        g = jax.nn.gelu(o)
        o_ref[...] = (jnp.dot(g, w_ref[...],
                              preferred_element_type=jnp.float32)
                      + b_ref[...] + s_ref[0, 0] * x_ref[...])

    return pl.pallas_call(
        body,
        grid=(grid,),
        in_specs=(
            [pl.BlockSpec((RB, HID), lambda i: (i, 0))]
            + [pl.BlockSpec((RB, HID), lambda i: (i, 0))] * nb
            + [pl.BlockSpec((HID, HID), lambda i: (0, 0)),
               pl.BlockSpec((1, HID), lambda i: (0, 0)),
               pl.BlockSpec(memory_space=pltpu.SMEM)]
        ),
        out_specs=pl.BlockSpec((RB, HID), lambda i: (i, 0)),
        out_shape=jax.ShapeDtypeStruct((M, HID), jnp.float32),
    )(x, *bufs, Wo, bo, sscal)


def kernel(x_author, x_paper, ei_writes, ei_rev_writes, ei_cites, params):
    preps = {
        "writes": _prep_edges(ei_writes),
        "rev_writes": _prep_edges(ei_rev_writes),
        "cites": _prep_edges(ei_cites),
    }
    rel_nt = {"writes": ("author", "paper"),
              "rev_writes": ("paper", "author"),
              "cites": ("paper", "paper")}
    x = {"author": x_author, "paper": x_paper}
    for lp in params:
        Wa, ba, Wp, bp, epi = _fold_layer(lp)
        qa, kv_w = _proj(x["author"], Wa, ba, [HID, 2 * HID])
        qp, kv_r, kv_c = _proj(x["paper"], Wp, bp, [HID, 2 * HID, 2 * HID])
        q = {"author": qa, "paper": qp}
        kv = {"writes": kv_w, "rev_writes": kv_r, "cites": kv_c}
        buf = {}
        for rel in ("writes", "rev_writes", "cites"):
            srcp, dstp, lseg3, first_seg, seg_of_node = preps[rel]
            _, dst_nt = rel_nt[rel]
            kve, qe = _edge_gather(kv[rel], q[dst_nt], srcp, dstp)
            segrows = _seg_softmax(kve, qe, lseg3, first_seg)
            buf[rel] = _seg_to_node(segrows, seg_of_node)
        x = {
            "author": _epilogue(x["author"], [buf["rev_writes"]], *epi["author"]),
            "paper": _epilogue(x["paper"], [buf["writes"], buf["cites"]],
                               *epi["paper"]),
        }
    return (x["author"], x["paper"])
